# baseline (speedup 1.0000x reference)
import functools

import jax
import jax.numpy as jnp
from jax import lax
from jax.experimental import pallas as pl
from jax.experimental.pallas import tpu as pltpu
from jax.experimental.pallas import tpu_sc as plsc

NC, NS = 2, 16
NW = NC * NS
L = 16
B, S, C = 4, 4096, 2048
ROWS = B * S
ROWS_PER_W = ROWS // NW
WPB = NW // B
CHUNK = 8
NIN = 4
NOUT = 2
NCHUNK = ROWS_PER_W // CHUNK
GROUPS = C // L


def _permute_rows(in_ref, out_ref, idx_v, r0, r1):
    @plsc.parallel_loop(0, GROUPS, unroll=4)
    def _(g):
        col = idx_v[pl.ds(g * L, L)]
        for r in range(r0, r1):
            row = jnp.full((L,), r, jnp.int32)
            v = plsc.load_gather(in_ref, [row, col])
            out_ref[r, pl.ds(g * L, L)] = v


def _shuffle_body(x_hbm, idx_hbm, out_hbm, idx_v, ins, outs, sis, sos):
    wid = lax.axis_index("s") * NC + lax.axis_index("c")
    b = wid // WPB
    row0 = (wid % WPB) * ROWS_PER_W

    def hbm_in(ci):
        return x_hbm.at[b, pl.ds(row0 + ci * CHUNK, CHUNK), :]

    def hbm_out(ci):
        return out_hbm.at[b, pl.ds(row0 + ci * CHUNK, CHUNK), :]

    for j in range(NIN):
        pltpu.async_copy(hbm_in(j), ins[j], sis[j])
    pltpu.sync_copy(idx_hbm, idx_v)

    nstep = NCHUNK // NIN

    def body(i, carry):
        for j in range(NIN):
            ci = NIN * i + j
            o = j % NOUT
            pltpu.make_async_copy(hbm_in(ci), ins[j], sis[j]).wait()

            if j < NOUT:
                @pl.when(i > 0)
                def _():
                    pltpu.make_async_copy(outs[o], hbm_out(ci), sos[o]).wait()
            else:
                pltpu.make_async_copy(outs[o], hbm_out(ci), sos[o]).wait()

            _permute_rows(ins[j], outs[o], idx_v, 0, CHUNK // 2)
            pltpu.async_copy(
                outs[o].at[pl.ds(0, CHUNK // 2), :],
                out_hbm.at[b, pl.ds(row0 + ci * CHUNK, CHUNK // 2), :],
                sos[o])
            _permute_rows(ins[j], outs[o], idx_v, CHUNK // 2, CHUNK)
            pltpu.async_copy(
                outs[o].at[pl.ds(CHUNK // 2, CHUNK // 2), :],
                out_hbm.at[b, pl.ds(row0 + ci * CHUNK + CHUNK // 2, CHUNK // 2), :],
                sos[o])

            @pl.when(i < nstep - 1)
            def _():
                pltpu.async_copy(hbm_in(ci + NIN), ins[j], sis[j])
        return carry

    lax.fori_loop(0, nstep, body, 0)
    for o in range(NOUT):
        pltpu.make_async_copy(outs[o], hbm_out(o), sos[o]).wait()


_shuffle = functools.partial(
    pl.kernel,
    out_type=jax.ShapeDtypeStruct((B, S, C), jnp.float32),
    mesh=plsc.VectorSubcoreMesh(
        core_axis_name="c", subcore_axis_name="s",
        num_cores=NC, num_subcores=NS,
    ),
    scratch_types=[
        pltpu.VMEM((C,), jnp.int32),
        tuple(pltpu.VMEM((CHUNK, C), jnp.float32) for _ in range(NIN)),
        tuple(pltpu.VMEM((CHUNK, C), jnp.float32) for _ in range(NOUT)),
        tuple(pltpu.SemaphoreType.DMA for _ in range(NIN)),
        tuple(pltpu.SemaphoreType.DMA for _ in range(NOUT)),
    ],
    compiler_params=pltpu.CompilerParams(needs_layout_passes=False),
)(_shuffle_body)


def kernel(x, forward_shuffle_idx):
    return _shuffle(x, forward_shuffle_idx)


# speedup vs baseline: 1.0179x; 1.0179x over previous
import functools

import jax
import jax.numpy as jnp
from jax import lax
from jax.experimental import pallas as pl
from jax.experimental.pallas import tpu as pltpu
from jax.experimental.pallas import tpu_sc as plsc

NC, NS = 2, 16
NW = NC * NS
L = 16
B, S, C = 4, 4096, 2048
ROWS = B * S
ROWS_PER_W = ROWS // NW
WPB = NW // B
CHUNK = 8
NIN = 4
NOUT = 2
NCHUNK = ROWS_PER_W // CHUNK
GROUPS = C // L


def _permute_chunk(in_ref, out_ref, idx_v):
    @plsc.parallel_loop(0, GROUPS, unroll=4)
    def _(g):
        col = idx_v[pl.ds(g * L, L)]
        for r in range(CHUNK):
            row = jnp.full((L,), r, jnp.int32)
            v = plsc.load_gather(in_ref, [row, col])
            out_ref[r, pl.ds(g * L, L)] = v


def _shuffle_body(x_hbm, idx_hbm, out_hbm, idx_v, ins, outs, sis, sos):
    wid = lax.axis_index("s") * NC + lax.axis_index("c")
    b = wid // WPB
    row0 = (wid % WPB) * ROWS_PER_W

    def hbm_in(ci):
        return x_hbm.at[b, pl.ds(row0 + ci * CHUNK, CHUNK), :]

    def hbm_out(ci):
        return out_hbm.at[b, pl.ds(row0 + ci * CHUNK, CHUNK), :]

    for j in range(NIN):
        pltpu.async_copy(hbm_in(j), ins[j], sis[j])
    pltpu.sync_copy(idx_hbm, idx_v)

    nstep = NCHUNK // NIN

    def body(i, carry):
        for j in range(NIN):
            ci = NIN * i + j
            o = j % NOUT
            pltpu.make_async_copy(hbm_in(ci), ins[j], sis[j]).wait()

            if j < NOUT:
                @pl.when(i > 0)
                def _():
                    pltpu.make_async_copy(outs[o], hbm_out(ci), sos[o]).wait()
            else:
                pltpu.make_async_copy(outs[o], hbm_out(ci), sos[o]).wait()

            _permute_chunk(ins[j], outs[o], idx_v)
            pltpu.async_copy(outs[o], hbm_out(ci), sos[o])

            @pl.when(i < nstep - 1)
            def _():
                pltpu.async_copy(hbm_in(ci + NIN), ins[j], sis[j])
        return carry

    lax.fori_loop(0, nstep, body, 0)
    for o in range(NOUT):
        pltpu.make_async_copy(outs[o], hbm_out(o), sos[o]).wait()


_shuffle = functools.partial(
    pl.kernel,
    out_type=jax.ShapeDtypeStruct((B, S, C), jnp.float32),
    mesh=plsc.VectorSubcoreMesh(
        core_axis_name="c", subcore_axis_name="s",
        num_cores=NC, num_subcores=NS,
    ),
    scratch_types=[
        pltpu.VMEM((C,), jnp.int32),
        tuple(pltpu.VMEM((CHUNK, C), jnp.float32) for _ in range(NIN)),
        tuple(pltpu.VMEM((CHUNK, C), jnp.float32) for _ in range(NOUT)),
        tuple(pltpu.SemaphoreType.DMA for _ in range(NIN)),
        tuple(pltpu.SemaphoreType.DMA for _ in range(NOUT)),
    ],
    compiler_params=pltpu.CompilerParams(needs_layout_passes=False, disable_bounds_checks=True, disable_semaphore_checks=True),
)(_shuffle_body)


def kernel(x, forward_shuffle_idx):
    return _shuffle(x, forward_shuffle_idx)


# speedup vs baseline: 1.0224x; 1.0044x over previous
import functools

import jax
import jax.numpy as jnp
from jax import lax
from jax.experimental import pallas as pl
from jax.experimental.pallas import tpu as pltpu
from jax.experimental.pallas import tpu_sc as plsc

NC, NS = 2, 16
NW = NC * NS
L = 16
B, S, C = 4, 4096, 2048
ROWS = B * S
ROWS_PER_W = ROWS // NW
WPB = NW // B
CHUNK = 8
NBUF = 3
NCHUNK = ROWS_PER_W // CHUNK
NSTEP = NCHUNK // NBUF
NTAIL = NCHUNK - NSTEP * NBUF
GROUPS = C // L


def _permute_chunk(in_ref, out_ref, idx_v):
    @plsc.parallel_loop(0, GROUPS, unroll=4)
    def _(g):
        col = idx_v[pl.ds(g * L, L)]
        for r in range(CHUNK):
            row = jnp.full((L,), r, jnp.int32)
            v = plsc.load_gather(in_ref, [row, col])
            out_ref[r, pl.ds(g * L, L)] = v


def _shuffle_body(x_hbm, idx_hbm, out_hbm, idx_v, ins, outs, sis, sos):
    wid = lax.axis_index("s") * NC + lax.axis_index("c")
    b = wid // WPB
    row0 = (wid % WPB) * ROWS_PER_W

    def hbm_in(ci):
        return x_hbm.at[b, pl.ds(row0 + ci * CHUNK, CHUNK), :]

    def hbm_out(ci):
        return out_hbm.at[b, pl.ds(row0 + ci * CHUNK, CHUNK), :]

    for j in range(NBUF):
        pltpu.async_copy(hbm_in(j), ins[j], sis[j])
    pltpu.sync_copy(idx_hbm, idx_v)

    def chunk_step(ci, j):
        pltpu.make_async_copy(hbm_in(ci), ins[j], sis[j]).wait()

        @pl.when(ci >= NBUF)
        def _():
            pltpu.make_async_copy(outs[j], hbm_out(ci), sos[j]).wait()

        _permute_chunk(ins[j], outs[j], idx_v)
        pltpu.async_copy(outs[j], hbm_out(ci), sos[j])

        @pl.when(ci + NBUF < NCHUNK)
        def _():
            pltpu.async_copy(hbm_in(ci + NBUF), ins[j], sis[j])

    def body(i, carry):
        for j in range(NBUF):
            chunk_step(NBUF * i + j, j)
        return carry

    lax.fori_loop(0, NSTEP, body, 0)
    for t in range(NTAIL):
        chunk_step(NSTEP * NBUF + t, t)
    for j in range(NBUF):
        pltpu.make_async_copy(outs[j], hbm_out(j), sos[j]).wait()


_shuffle = functools.partial(
    pl.kernel,
    out_type=jax.ShapeDtypeStruct((B, S, C), jnp.float32),
    mesh=plsc.VectorSubcoreMesh(
        core_axis_name="c", subcore_axis_name="s",
        num_cores=NC, num_subcores=NS,
    ),
    scratch_types=[
        pltpu.VMEM((C,), jnp.int32),
        tuple(pltpu.VMEM((CHUNK, C), jnp.float32) for _ in range(NBUF)),
        tuple(pltpu.VMEM((CHUNK, C), jnp.float32) for _ in range(NBUF)),
        tuple(pltpu.SemaphoreType.DMA for _ in range(NBUF)),
        tuple(pltpu.SemaphoreType.DMA for _ in range(NBUF)),
    ],
    compiler_params=pltpu.CompilerParams(needs_layout_passes=False),
)(_shuffle_body)


def kernel(x, forward_shuffle_idx):
    return _shuffle(x, forward_shuffle_idx)
